# baseline (speedup 1.0000x reference)
import functools

import jax
import jax.numpy as jnp
from jax.experimental import pallas as pl
from jax.experimental.pallas import tpu as pltpu

N_NODES = 10000
D_FEAT = 128
TM = 80


def _matmul_block(adj_ref, emb_ref, out_ref):
    out_ref[...] = jnp.dot(
        adj_ref[...], emb_ref[...], preferred_element_type=jnp.float32
    )


@functools.partial(jax.jit, static_argnames=())
def kernel(adj, embeds):
    grid = (N_NODES // TM,)
    return pl.pallas_call(
        _matmul_block,
        grid=grid,
        in_specs=[
            pl.BlockSpec((TM, N_NODES), lambda i: (i, 0)),
            pl.BlockSpec((N_NODES, D_FEAT), lambda i: (0, 0)),
        ],
        out_specs=pl.BlockSpec((TM, D_FEAT), lambda i: (i, 0)),
        out_shape=jax.ShapeDtypeStruct((N_NODES, D_FEAT), jnp.float32),
        compiler_params=pltpu.CompilerParams(
            dimension_semantics=("parallel",),
        ),
    )(adj, embeds)


# speedup vs baseline: 1.3825x; 1.3825x over previous
import functools

import jax
import jax.numpy as jnp
from jax.experimental import pallas as pl
from jax.experimental.pallas import tpu as pltpu

N_NODES = 10000
D_FEAT = 128
TM = 200


def _matmul_block(adj_ref, emb_ref, out_ref):
    out_ref[...] = jnp.dot(
        adj_ref[...], emb_ref[...], preferred_element_type=jnp.float32
    )


@functools.partial(jax.jit, static_argnames=())
def kernel(adj, embeds):
    grid = (N_NODES // TM,)
    return pl.pallas_call(
        _matmul_block,
        grid=grid,
        in_specs=[
            pl.BlockSpec((TM, N_NODES), lambda i: (i, 0)),
            pl.BlockSpec((N_NODES, D_FEAT), lambda i: (0, 0)),
        ],
        out_specs=pl.BlockSpec((TM, D_FEAT), lambda i: (i, 0)),
        out_shape=jax.ShapeDtypeStruct((N_NODES, D_FEAT), jnp.float32),
        compiler_params=pltpu.CompilerParams(
            dimension_semantics=("parallel",),
        ),
    )(adj, embeds)


# speedup vs baseline: 1.3901x; 1.0055x over previous
import functools

import jax
import jax.numpy as jnp
from jax.experimental import pallas as pl
from jax.experimental.pallas import tpu as pltpu

N_NODES = 10000
D_FEAT = 128
TM = 80
NBUF = 4


def _outer(adj_hbm, emb_ref, out_hbm):
    def _inner(adj_blk, out_blk):
        out_blk[...] = jnp.dot(
            adj_blk[...], emb_ref[...], preferred_element_type=jnp.float32
        )

    pltpu.emit_pipeline(
        _inner,
        grid=(N_NODES // TM,),
        in_specs=[
            pl.BlockSpec(
                (TM, N_NODES),
                lambda i: (i, 0),
                pipeline_mode=pl.Buffered(buffer_count=NBUF),
            )
        ],
        out_specs=[pl.BlockSpec((TM, D_FEAT), lambda i: (i, 0))],
    )(adj_hbm, out_hbm)


@functools.partial(jax.jit, static_argnames=())
def kernel(adj, embeds):
    return pl.pallas_call(
        _outer,
        in_specs=[
            pl.BlockSpec(memory_space=pltpu.HBM),
            pl.BlockSpec((N_NODES, D_FEAT), lambda: (0, 0)),
        ],
        out_specs=pl.BlockSpec(memory_space=pltpu.HBM),
        out_shape=jax.ShapeDtypeStruct((N_NODES, D_FEAT), jnp.float32),
    )(adj, embeds)
